# baseline (speedup 1.0000x reference)
import functools

import jax
import jax.numpy as jnp
from jax import lax
from jax.experimental import pallas as pl
from jax.experimental.pallas import tpu as pltpu
from jax.experimental.pallas import tpu_sc as plsc

B = 16384
D = 128
N = 100000
NC = 2
NS = 16
NW = NC * NS
RS = 13
W16 = 1 << RS
PW = B // NW
SUB = PW // 128
ROUNDS = 0

_mesh = plsc.VectorSubcoreMesh(core_axis_name="c", subcore_axis_name="s")


@functools.partial(
    pl.kernel,
    out_type=jax.ShapeDtypeStruct((B, D), jnp.float32),
    mesh=_mesh,
    scratch_types=[
        pltpu.VMEM((B,), jnp.int32),
        pltpu.VMEM((W16,), jnp.int32),
        pltpu.VMEM((SUB, 128), jnp.int32),
        pltpu.VMEM((PW,), jnp.int32),
        pltpu.VMEM((PW, D), jnp.float32),
        pltpu.VMEM((SUB, 128), jnp.int32),
        pltpu.VMEM((SUB, 128), jnp.int32),
        pltpu.VMEM((128, D), jnp.float32),
        pltpu.VMEM_SHARED((NS * W16,), jnp.int32),
        pltpu.SemaphoreType.DMA,
        pltpu.SemaphoreType.DMA,
    ],
    compiler_params=pltpu.CompilerParams(needs_layout_passes=False),
)
def _push_pull(x_hbm, push_hbm, pull_hbm, emb_hbm, out_hbm,
               push_v, tag_v, pull_v, t_v, rows_v, xsrc_v, xpos_v, xr_v,
               tag_s, sem, sem2):
    c = lax.axis_index("c")
    s = lax.axis_index("s")
    w = s * NC + c
    iota16 = lax.broadcasted_iota(jnp.int32, (16,), 0)
    zeros16 = jnp.zeros((16,), jnp.int32)

    with jax.named_scope("ph_stage"):
        pltpu.sync_copy(pull_hbm.at[w], pull_v)
        gathers = [
            pltpu.async_copy(emb_hbm.at[pull_v.at[j]],
                             rows_v.at[pl.ds(j * 128, 128)], sem)
            for j in range(SUB)
        ]
        push_cp = pltpu.async_copy(push_hbm, push_v, sem2)

        def _zero(i):
            tag_v[pl.ds(i * 16, 16)] = zeros16

        plsc.parallel_loop(0, W16 // 16, unroll=8)(_zero)

        push_cp.wait()

    def _scan(g, carry):
        idx = push_v[pl.ds(g * 16, 16)]
        m = (idx >> RS) == s
        loc = idx & (W16 - 1)
        jv = iota16 + (g * 16 + 1)
        if ROUNDS == 0:
            plsc.store_scatter(tag_v, [loc], jv, mask=m)
        for _ in range(ROUNDS):
            cur = plsc.load_gather(tag_v, [loc])
            plsc.store_scatter(tag_v, [loc], jv, mask=m & (jv > cur))
        return carry

    with jax.named_scope("ph_scan"):
        lax.fori_loop(0, B // 16, _scan, 0, unroll=4)

    with jax.named_scope("ph_publish"):
        pltpu.sync_copy(tag_v, tag_s.at[pl.ds(s * W16, W16)])
        plsc.subcore_barrier()

    with jax.named_scope("ph_tgather"):
        tg = [
            pltpu.async_copy(tag_s.at[pull_v.at[j]],
                             t_v.at[pl.ds(j * 128, 128)], sem2)
            for j in range(SUB)
        ]
        out_cps = []
        for j in range(SUB):
            gathers[j].wait()
            out_cps.append(
                pltpu.async_copy(rows_v.at[pl.ds(j * 128, 128)],
                                 out_hbm.at[pl.ds(w * PW + j * 128, 128)],
                                 sem))
        for h in tg:
            h.wait()

    with jax.named_scope("ph_compact"):
        off = jnp.int32(0)
        for g in range(PW // 16):
            tv = t_v[pl.ds(g * 16, 16)]
            m = tv > 0
            inc = plsc.cumsum(jnp.where(m, 1, 0))
            addr = jnp.maximum(off + inc - 1, 0)
            a_hi = addr >> 7
            a_lo = addr & 127
            plsc.store_scatter(xsrc_v, [a_hi, a_lo], tv - 1, mask=m)
            plsc.store_scatter(xpos_v, [a_hi, a_lo],
                               iota16 + (w * PW + g * 16), mask=m)
            off = off + jnp.sum(jnp.where(m, 1, 0))

    nh = off
    trips = (nh + 127) >> 7

    s0 = plsc.load_gather(xsrc_v, [zeros16, zeros16])
    p0 = plsc.load_gather(xpos_v, [zeros16, zeros16])

    def _fill(g, carry):
        pos = iota16 + g * 16
        row = g >> 3
        col = (g & 7) * 16
        mfill = pos >= nh
        cs = xsrc_v[row, pl.ds(col, 16)]
        cp = xpos_v[row, pl.ds(col, 16)]
        xsrc_v[row, pl.ds(col, 16)] = jnp.where(mfill, s0, cs)
        xpos_v[row, pl.ds(col, 16)] = jnp.where(mfill, p0, cp)
        return carry

    with jax.named_scope("ph_fill"):
        lax.fori_loop(nh >> 4, trips << 3, _fill, 0)

    with jax.named_scope("ph_outwait"):
        for h in out_cps:
            h.wait()

    with jax.named_scope("ph_hits"):
        def _hits(k, carry):
            pltpu.sync_copy(x_hbm.at[xsrc_v.at[k]], xr_v)
            pltpu.sync_copy(xr_v, out_hbm.at[xpos_v.at[k]])
            return carry

        lax.fori_loop(0, trips, _hits, 0)


def kernel(x, push_inds, pull_inds, emb):
    push_i = push_inds.astype(jnp.int32)
    pull_i = pull_inds.astype(jnp.int32).reshape(NW, SUB, 128)
    return _push_pull(x, push_i, pull_i, emb)


# speedup vs baseline: 1.1894x; 1.1894x over previous
import functools

import jax
import jax.numpy as jnp
from jax import lax
from jax.experimental import pallas as pl
from jax.experimental.pallas import tpu as pltpu
from jax.experimental.pallas import tpu_sc as plsc

B = 16384
D = 128
N = 100000
NC = 2
NS = 16
NW = NC * NS
RS = 13
W16 = 1 << RS
PW = B // NW
SUB = PW // 128
ROUNDS = 0

_mesh = plsc.VectorSubcoreMesh(core_axis_name="c", subcore_axis_name="s")


@functools.partial(
    pl.kernel,
    out_type=jax.ShapeDtypeStruct((B, D), jnp.float32),
    mesh=_mesh,
    scratch_types=[
        pltpu.VMEM((B + 16,), jnp.int32),
        pltpu.VMEM((W16,), jnp.int32),
        pltpu.VMEM((SUB, 128), jnp.int32),
        pltpu.VMEM((PW,), jnp.int32),
        pltpu.VMEM((PW, D), jnp.float32),
        pltpu.VMEM((SUB, 128), jnp.int32),
        pltpu.VMEM((SUB, 128), jnp.int32),
        pltpu.VMEM((128, D), jnp.float32),
        pltpu.VMEM_SHARED((NS * W16,), jnp.int32),
        pltpu.SemaphoreType.DMA,
        pltpu.SemaphoreType.DMA,
    ],
    compiler_params=pltpu.CompilerParams(needs_layout_passes=False),
)
def _push_pull(x_hbm, push_hbm, pull_hbm, emb_hbm, out_hbm,
               push_v, tag_v, pull_v, t_v, rows_v, xsrc_v, xpos_v, xr_v,
               tag_s, sem, sem2):
    c = lax.axis_index("c")
    s = lax.axis_index("s")
    w = s * NC + c
    iota16 = lax.broadcasted_iota(jnp.int32, (16,), 0)
    zeros16 = jnp.zeros((16,), jnp.int32)

    with jax.named_scope("ph_stage"):
        push_cp = pltpu.async_copy(push_hbm, push_v.at[pl.ds(0, B)], sem2)
        pltpu.sync_copy(pull_hbm.at[w], pull_v)
        gathers = [
            pltpu.async_copy(emb_hbm.at[pull_v.at[j]],
                             rows_v.at[pl.ds(j * 128, 128)], sem)
            for j in range(SUB)
        ]

        def _zero(i):
            tag_v[pl.ds(i * 16, 16)] = zeros16

        plsc.parallel_loop(0, W16 // 16, unroll=8)(_zero)

        push_cp.wait()

    def _scan(g, idx):
        nxt = push_v[pl.ds(g * 16 + 16, 16)]
        m = (idx >> RS) == s
        loc = idx & (W16 - 1)
        jv = iota16 + (g * 16 + 1)
        if ROUNDS == 0:
            plsc.store_scatter(tag_v, [loc], jv, mask=m)
        for _ in range(ROUNDS):
            cur = plsc.load_gather(tag_v, [loc])
            plsc.store_scatter(tag_v, [loc], jv, mask=m & (jv > cur))
        return nxt

    with jax.named_scope("ph_scan"):
        idx0 = push_v[pl.ds(0, 16)]
        lax.fori_loop(0, B // 16, _scan, idx0, unroll=4)

    with jax.named_scope("ph_publish"):
        pltpu.sync_copy(tag_v, tag_s.at[pl.ds(s * W16, W16)])
        plsc.subcore_barrier()

    with jax.named_scope("ph_tgather"):
        tg = [
            pltpu.async_copy(tag_s.at[pull_v.at[j]],
                             t_v.at[pl.ds(j * 128, 128)], sem2)
            for j in range(SUB)
        ]
        out_cps = []
        for j in range(SUB):
            gathers[j].wait()
            out_cps.append(
                pltpu.async_copy(rows_v.at[pl.ds(j * 128, 128)],
                                 out_hbm.at[pl.ds(w * PW + j * 128, 128)],
                                 sem))
        for h in tg:
            h.wait()

    with jax.named_scope("ph_compact"):
        off = jnp.int32(0)
        for g in range(PW // 16):
            tv = t_v[pl.ds(g * 16, 16)]
            m = tv > 0
            inc = plsc.cumsum(jnp.where(m, 1, 0))
            addr = jnp.maximum(off + inc - 1, 0)
            a_hi = addr >> 7
            a_lo = addr & 127
            plsc.store_scatter(xsrc_v, [a_hi, a_lo], tv - 1, mask=m)
            plsc.store_scatter(xpos_v, [a_hi, a_lo],
                               iota16 + (w * PW + g * 16), mask=m)
            off = off + jnp.sum(jnp.where(m, 1, 0))

    nh = off
    trips = (nh + 127) >> 7

    s0 = plsc.load_gather(xsrc_v, [zeros16, zeros16])
    p0 = plsc.load_gather(xpos_v, [zeros16, zeros16])

    def _fill(g, carry):
        pos = iota16 + g * 16
        row = g >> 3
        col = (g & 7) * 16
        mfill = pos >= nh
        cs = xsrc_v[row, pl.ds(col, 16)]
        cp = xpos_v[row, pl.ds(col, 16)]
        xsrc_v[row, pl.ds(col, 16)] = jnp.where(mfill, s0, cs)
        xpos_v[row, pl.ds(col, 16)] = jnp.where(mfill, p0, cp)
        return carry

    with jax.named_scope("ph_fill"):
        lax.fori_loop(nh >> 4, trips << 3, _fill, 0)

    @pl.when(trips > 0)
    def _pre():
        pltpu.sync_copy(x_hbm.at[xsrc_v.at[0]], xr_v)

    with jax.named_scope("ph_outwait"):
        for h in out_cps:
            h.wait()

    with jax.named_scope("ph_hits"):
        @pl.when(trips > 0)
        def _sc0():
            pltpu.sync_copy(xr_v, out_hbm.at[xpos_v.at[0]])

        def _hits(k, carry):
            pltpu.sync_copy(x_hbm.at[xsrc_v.at[k]], xr_v)
            pltpu.sync_copy(xr_v, out_hbm.at[xpos_v.at[k]])
            return carry

        lax.fori_loop(1, trips, _hits, 0)


def kernel(x, push_inds, pull_inds, emb):
    push_i = push_inds.astype(jnp.int32)
    pull_i = pull_inds.astype(jnp.int32).reshape(NW, SUB, 128)
    return _push_pull(x, push_i, pull_i, emb)


# speedup vs baseline: 1.2026x; 1.0111x over previous
import functools

import jax
import jax.numpy as jnp
from jax import lax
from jax.experimental import pallas as pl
from jax.experimental.pallas import tpu as pltpu
from jax.experimental.pallas import tpu_sc as plsc

B = 16384
D = 128
N = 100000
NC = 2
NS = 16
NW = NC * NS
RS = 13
W16 = 1 << RS
PW = B // NW
SUB = PW // 128

_mesh = plsc.VectorSubcoreMesh(core_axis_name="c", subcore_axis_name="s")


@functools.partial(
    pl.kernel,
    out_type=jax.ShapeDtypeStruct((B, D), jnp.float32),
    mesh=_mesh,
    scratch_types=[
        pltpu.VMEM((B + 16,), jnp.int32),
        pltpu.VMEM((W16,), jnp.int32),
        pltpu.VMEM((SUB, 128), jnp.int32),
        pltpu.VMEM((PW,), jnp.int32),
        pltpu.VMEM((PW, D), jnp.float32),
        pltpu.VMEM((SUB, 128), jnp.int32),
        pltpu.VMEM((SUB, 128), jnp.int32),
        pltpu.VMEM((128, D), jnp.float32),
        pltpu.VMEM_SHARED((NS * W16,), jnp.int32),
        pltpu.SemaphoreType.DMA,
        pltpu.SemaphoreType.DMA,
    ],
    compiler_params=pltpu.CompilerParams(needs_layout_passes=False),
)
def _push_pull(x_hbm, push_hbm, pull_hbm, emb_hbm, out_hbm,
               push_v, tag_v, pull_v, t_v, rows_v, xsrc_v, xpos_v, xr_v,
               tag_s, sem, sem2):
    c = lax.axis_index("c")
    s = lax.axis_index("s")
    w = s * NC + c
    iota16 = lax.broadcasted_iota(jnp.int32, (16,), 0)
    zeros16 = jnp.zeros((16,), jnp.int32)

    push_cp = pltpu.async_copy(push_hbm, push_v.at[pl.ds(0, B)], sem2)
    pull_cp = pltpu.async_copy(pull_hbm.at[w], pull_v, sem)

    def _zero(i):
        tag_v[pl.ds(i * 16, 16)] = zeros16

    plsc.parallel_loop(0, W16 // 16, unroll=8)(_zero)

    pull_cp.wait()
    gathers = [
        pltpu.async_copy(emb_hbm.at[pull_v.at[j]],
                         rows_v.at[pl.ds(j * 128, 128)], sem)
        for j in range(SUB)
    ]
    push_cp.wait()

    base = s * W16

    def _scan(g, idx):
        nxt = push_v[pl.ds(g * 16 + 16, 16)]
        m = (idx >> RS) == s
        loc = idx & (W16 - 1)
        jv = iota16 + (g * 16 + 1)
        plsc.store_scatter(tag_v, [loc], jv, mask=m)
        return nxt

    lax.fori_loop(0, B // 16, _scan, push_v[pl.ds(0, 16)], unroll=8)

    pltpu.sync_copy(tag_v, tag_s.at[pl.ds(base, W16)])
    plsc.subcore_barrier()

    tg = [
        pltpu.async_copy(tag_s.at[pull_v.at[j]],
                         t_v.at[pl.ds(j * 128, 128)], sem2)
        for j in range(SUB)
    ]
    out_cps = []
    for j in range(SUB):
        gathers[j].wait()
        out_cps.append(
            pltpu.async_copy(rows_v.at[pl.ds(j * 128, 128)],
                             out_hbm.at[pl.ds(w * PW + j * 128, 128)], sem))
    for h in tg:
        h.wait()

    off = jnp.int32(0)
    for g in range(PW // 16):
        tv = t_v[pl.ds(g * 16, 16)]
        m = tv > 0
        inc = plsc.cumsum(jnp.where(m, 1, 0))
        addr = jnp.maximum(off + inc - 1, 0)
        a_hi = addr >> 7
        a_lo = addr & 127
        plsc.store_scatter(xsrc_v, [a_hi, a_lo], tv - 1, mask=m)
        plsc.store_scatter(xpos_v, [a_hi, a_lo], iota16 + (w * PW + g * 16),
                           mask=m)
        off = off + jnp.sum(jnp.where(m, 1, 0))

    nh = off
    trips = (nh + 127) >> 7

    s0 = plsc.load_gather(xsrc_v, [zeros16, zeros16])
    p0 = plsc.load_gather(xpos_v, [zeros16, zeros16])

    def _fill(g, carry):
        pos = iota16 + g * 16
        row = g >> 3
        col = (g & 7) * 16
        mfill = pos >= nh
        cs = xsrc_v[row, pl.ds(col, 16)]
        cp = xpos_v[row, pl.ds(col, 16)]
        xsrc_v[row, pl.ds(col, 16)] = jnp.where(mfill, s0, cs)
        xpos_v[row, pl.ds(col, 16)] = jnp.where(mfill, p0, cp)
        return carry

    lax.fori_loop(nh >> 4, trips << 3, _fill, 0)

    @pl.when(trips > 0)
    def _pre():
        pltpu.sync_copy(x_hbm.at[xsrc_v.at[0]], xr_v)

    for h in out_cps:
        h.wait()

    @pl.when(trips > 0)
    def _sc0():
        pltpu.sync_copy(xr_v, out_hbm.at[xpos_v.at[0]])

    def _hits(k, carry):
        pltpu.sync_copy(x_hbm.at[xsrc_v.at[k]], xr_v)
        pltpu.sync_copy(xr_v, out_hbm.at[xpos_v.at[k]])
        return carry

    lax.fori_loop(1, trips, _hits, 0)


def kernel(x, push_inds, pull_inds, emb):
    push_i = push_inds.astype(jnp.int32)
    pull_i = pull_inds.astype(jnp.int32).reshape(NW, SUB, 128)
    return _push_pull(x, push_i, pull_i, emb)
